# baseline (speedup 1.0000x reference)
import functools

import jax
import jax.numpy as jnp
from jax import lax
from jax.experimental import pallas as pl
from jax.experimental.pallas import tpu as pltpu
from jax.experimental.pallas import tpu_sc as plsc

_NC = 2
_NS = 16
_L = 16
_CHUNK = 128
_OH = 16


@functools.lru_cache(maxsize=None)
def _make_gather(V, D, B):
    NW = _NC * _NS
    b_per_w = B // NW
    n_chunks = b_per_w // _CHUNK
    assert b_per_w * NW == B and n_chunks * _CHUNK == b_per_w
    W = 2 * D
    mesh = plsc.VectorSubcoreMesh(core_axis_name="c", subcore_axis_name="s")

    @functools.partial(
        pl.kernel,
        mesh=mesh,
        compiler_params=pltpu.CompilerParams(use_tc_tiling_on_sc=False,
                                             needs_layout_passes=False),
        out_type=[
            jax.ShapeDtypeStruct((B, W), jnp.float32),
            jax.ShapeDtypeStruct((B, W), jnp.float32),
        ],
        scratch_types=[
            pltpu.VMEM((b_per_w,), jnp.int32),
            pltpu.VMEM((b_per_w,), jnp.int32),
            pltpu.VMEM((b_per_w,), jnp.int32),
            pltpu.VMEM((b_per_w, D), jnp.float32),
            pltpu.VMEM((b_per_w, D), jnp.float32),
            pltpu.VMEM((b_per_w, _OH), jnp.float32),
            pltpu.SemaphoreType.DMA,
        ],
    )
    def gather(table_hbm, nidx_hbm, midx_hbm, ridx_hbm, out_n, out_m,
               idx_n, idx_m, rel_v, rows_n, rows_m, oh, sem):
        wid = lax.axis_index("s") * _NC + lax.axis_index("c")
        base = wid * b_per_w
        pltpu.sync_copy(nidx_hbm.at[pl.ds(base, b_per_w)], idx_n)
        pltpu.sync_copy(midx_hbm.at[pl.ds(base, b_per_w)], idx_m)
        pltpu.sync_copy(ridx_hbm.at[pl.ds(base, b_per_w)], rel_v)
        copies = []
        for j in range(n_chunks):
            sl = pl.ds(j * _CHUNK, _CHUNK)
            copies.append(pltpu.async_copy(
                table_hbm.at[idx_n.at[sl]], rows_n.at[sl], sem))
            copies.append(pltpu.async_copy(
                table_hbm.at[idx_m.at[sl]], rows_m.at[sl], sem))

        zeros = jnp.zeros((_L,), jnp.float32)

        def zbody(i, _):
            oh[i, :] = zeros
            return ()

        lax.fori_loop(0, b_per_w, zbody, ())
        ones = jnp.full((_L,), 1.0, jnp.float32)
        iota = lax.iota(jnp.int32, _L)

        def sbody(i, _):
            rows = iota + i * _L
            cols = rel_v[pl.ds(i * _L, _L)]
            plsc.store_scatter(oh, [rows, cols], ones)
            return ()

        lax.fori_loop(0, b_per_w // _L, sbody, ())

        for c in copies:
            c.wait()
        rows = pl.ds(base, b_per_w)
        pltpu.sync_copy(rows_n, out_n.at[rows, pl.ds(0, D)])
        pltpu.sync_copy(oh, out_n.at[rows, pl.ds(D, _OH)])
        pltpu.sync_copy(rows_m, out_m.at[rows, pl.ds(0, D)])

    return gather


def _dot(a, b):
    return lax.dot_general(a, b, (((1,), (0,)), ((), ())),
                           preferred_element_type=jnp.float32)


def _score_body(nrel, node_ref, nbr_ref, rv_ref, out_ref):
    D = rv_ref.shape[1]
    RD = nrel * D
    nw = node_ref[...]
    node = nw[:, :D]
    oh = nw[:, D:D + nrel]
    nbr = nbr_ref[:, :D]
    rv = rv_ref[...]
    rh = jnp.concatenate([rv[r * D:(r + 1) * D, :] for r in range(nrel)],
                         axis=1)
    t8 = _dot(node, rh)
    li = lax.broadcasted_iota(jnp.int32, (D, RD), 1)
    ri = lax.broadcasted_iota(jnp.int32, (D, RD), 0)
    ti = (li % D == ri).astype(jnp.float32)
    nbr8 = _dot(nbr, ti)
    p8 = t8 * nbr8
    rr = lax.broadcasted_iota(jnp.int32, (RD, nrel), 0)
    cc = lax.broadcasted_iota(jnp.int32, (RD, nrel), 1)
    s8 = (rr // D == cc).astype(jnp.float32)
    q = _dot(p8, s8)
    ones = jnp.ones((nrel, 1), jnp.float32)
    score = _dot(q * oh, ones)
    out_ref[...] = jax.nn.sigmoid(score)


@functools.lru_cache(maxsize=None)
def _make_score(B, D, R, Bb=4096, interpret=False):
    grid = (B // Bb,)
    return pl.pallas_call(
        functools.partial(_score_body, R),
        grid=grid,
        in_specs=[
            pl.BlockSpec((Bb, 2 * D), lambda i: (i, 0)),
            pl.BlockSpec((Bb, 2 * D), lambda i: (i, 0)),
            pl.BlockSpec((R * D, D), lambda i: (0, 0)),
        ],
        out_specs=pl.BlockSpec((Bb, 1), lambda i: (i, 0)),
        out_shape=jax.ShapeDtypeStruct((B, 1), jnp.float32),
        interpret=interpret,
    )


def kernel(node_idx, relation_idx, node_neighbor_idx, node_embed_table,
           relation_embed_table):
    B = node_idx.shape[0]
    V, D = node_embed_table.shape
    R = relation_embed_table.shape[0]
    rv = relation_embed_table.reshape(R * D, D)
    nidx = node_idx.astype(jnp.int32)
    midx = node_neighbor_idx.astype(jnp.int32)
    ridx = relation_idx.astype(jnp.int32)
    H = B // 2
    gather = _make_gather(V, D, H)
    score = _make_score(H, D, R)
    outs = []
    rows = [gather(node_embed_table, nidx[h * H:(h + 1) * H],
                   midx[h * H:(h + 1) * H], ridx[h * H:(h + 1) * H])
            for h in range(2)]
    for h in range(2):
        outs.append(score(rows[h][0], rows[h][1], rv))
    return jnp.concatenate(outs, axis=0)


# speedup vs baseline: 1.0075x; 1.0075x over previous
import functools

import jax
import jax.numpy as jnp
from jax import lax
from jax.experimental import pallas as pl
from jax.experimental.pallas import tpu as pltpu
from jax.experimental.pallas import tpu_sc as plsc

_NC = 2
_NS = 16
_L = 16
_CHUNK = 128
_OH = 16


@functools.lru_cache(maxsize=None)
def _make_gather(V, D, B):
    NW = _NC * _NS
    b_per_w = B // NW
    n_chunks = b_per_w // _CHUNK
    assert b_per_w * NW == B and n_chunks * _CHUNK == b_per_w
    W = 2 * D
    mesh = plsc.VectorSubcoreMesh(core_axis_name="c", subcore_axis_name="s")

    @functools.partial(
        pl.kernel,
        mesh=mesh,
        compiler_params=pltpu.CompilerParams(use_tc_tiling_on_sc=False,
                                             needs_layout_passes=False),
        out_type=[
            jax.ShapeDtypeStruct((B, W), jnp.float32),
            jax.ShapeDtypeStruct((B, W), jnp.float32),
        ],
        scratch_types=[
            pltpu.VMEM((b_per_w,), jnp.int32),
            pltpu.VMEM((b_per_w,), jnp.int32),
            pltpu.VMEM((b_per_w,), jnp.int32),
            pltpu.VMEM((b_per_w, D), jnp.float32),
            pltpu.VMEM((b_per_w, D), jnp.float32),
            pltpu.VMEM((b_per_w, _OH), jnp.float32),
            pltpu.SemaphoreType.DMA,
        ],
    )
    def gather(table_hbm, nidx_hbm, midx_hbm, ridx_hbm, out_n, out_m,
               idx_n, idx_m, rel_v, rows_n, rows_m, oh, sem):
        wid = lax.axis_index("s") * _NC + lax.axis_index("c")
        base = wid * b_per_w
        pltpu.sync_copy(nidx_hbm.at[pl.ds(base, b_per_w)], idx_n)
        pltpu.sync_copy(midx_hbm.at[pl.ds(base, b_per_w)], idx_m)
        pltpu.sync_copy(ridx_hbm.at[pl.ds(base, b_per_w)], rel_v)
        copies = []
        for j in range(n_chunks):
            sl = pl.ds(j * _CHUNK, _CHUNK)
            copies.append(pltpu.async_copy(
                table_hbm.at[idx_n.at[sl]], rows_n.at[sl], sem))
            copies.append(pltpu.async_copy(
                table_hbm.at[idx_m.at[sl]], rows_m.at[sl], sem))

        zeros = jnp.zeros((_L,), jnp.float32)

        def zbody(i, _):
            oh[i, :] = zeros
            return ()

        lax.fori_loop(0, b_per_w, zbody, ())
        ones = jnp.full((_L,), 1.0, jnp.float32)
        iota = lax.iota(jnp.int32, _L)

        def sbody(i, _):
            rows = iota + i * _L
            cols = rel_v[pl.ds(i * _L, _L)]
            plsc.store_scatter(oh, [rows, cols], ones)
            return ()

        lax.fori_loop(0, b_per_w // _L, sbody, ())

        for c in copies:
            c.wait()
        rows = pl.ds(base, b_per_w)
        pltpu.sync_copy(rows_n, out_n.at[rows, pl.ds(0, D)])
        pltpu.sync_copy(oh, out_n.at[rows, pl.ds(D, _OH)])
        pltpu.sync_copy(rows_m, out_m.at[rows, pl.ds(0, D)])

    return gather


def _dot(a, b):
    return lax.dot_general(a, b, (((1,), (0,)), ((), ())),
                           preferred_element_type=jnp.float32)


def _score_body(nrel, node_ref, nbr_ref, rv_ref, out_ref):
    D = rv_ref.shape[1]
    RD = nrel * D
    nw = node_ref[...]
    node = nw[:, :D]
    oh = nw[:, D:D + nrel]
    nbr = nbr_ref[:, :D]
    rv = rv_ref[...]
    rh = jnp.concatenate([rv[r * D:(r + 1) * D, :] for r in range(nrel)],
                         axis=1)
    t8 = _dot(node, rh)
    li = lax.broadcasted_iota(jnp.int32, (D, RD), 1)
    ri = lax.broadcasted_iota(jnp.int32, (D, RD), 0)
    ti = (li % D == ri).astype(jnp.float32)
    nbr8 = _dot(nbr, ti)
    p8 = t8 * nbr8
    rr = lax.broadcasted_iota(jnp.int32, (RD, nrel), 0)
    cc = lax.broadcasted_iota(jnp.int32, (RD, nrel), 1)
    s8 = (rr // D == cc).astype(jnp.float32)
    q = _dot(p8, s8)
    ones = jnp.ones((nrel, 1), jnp.float32)
    score = _dot(q * oh, ones)
    out_ref[...] = jax.nn.sigmoid(score)


@functools.lru_cache(maxsize=None)
def _make_score(B, D, R, Bb=4096, interpret=False):
    grid = (B // Bb,)
    return pl.pallas_call(
        functools.partial(_score_body, R),
        grid=grid,
        in_specs=[
            pl.BlockSpec((Bb, 2 * D), lambda i: (i, 0)),
            pl.BlockSpec((Bb, 2 * D), lambda i: (i, 0)),
            pl.BlockSpec((R * D, D), lambda i: (0, 0)),
        ],
        out_specs=pl.BlockSpec((Bb, 1), lambda i: (i, 0)),
        out_shape=jax.ShapeDtypeStruct((B, 1), jnp.float32),
        interpret=interpret,
    )


def kernel(node_idx, relation_idx, node_neighbor_idx, node_embed_table,
           relation_embed_table):
    B = node_idx.shape[0]
    V, D = node_embed_table.shape
    R = relation_embed_table.shape[0]
    node_rows, nbr_rows = _make_gather(V, D, B)(
        node_embed_table, node_idx.astype(jnp.int32),
        node_neighbor_idx.astype(jnp.int32), relation_idx.astype(jnp.int32))
    rv = relation_embed_table.reshape(R * D, D)
    return _make_score(B, D, R)(node_rows, nbr_rows, rv)
